# Initial kernel scaffold; baseline (speedup 1.0000x reference)
import jax
import jax.numpy as jnp
from jax.experimental import pallas as pl


def kernel(nodes, edges, senders, receivers, cutoff, params):
    raise NotImplementedError("write your pallas kernel here")


# speedup vs baseline: 1.8093x; 1.8093x over previous
import functools

import jax
import jax.numpy as jnp
from jax import lax
from jax.experimental import pallas as pl
from jax.experimental.pallas import tpu as pltpu
from jax.experimental.pallas import tpu_sc as plsc

N = 10000
E = 320000
D = 128

NC = 2
NS = 16
CHUNK = 128
NCHUNKS = E // CHUNK
NJ = NCHUNKS // NS
EXTRA = NCHUNKS % NS
ROWS_PER_TILE = 624
ROWS_REM = N - NS * ROWS_PER_TILE

_LOG2 = 0.6931471805599453


def _gather_body(nodes_hbm, send_hbm, recv_hbm, sent_out, recv_out,
                 idx_v, rows_v, sem):
    c = lax.axis_index("c")
    s = lax.axis_index("s")
    nj = jnp.where(s < EXTRA, NJ + 1, NJ)

    def run(idx_hbm, out_hbm):
        def body(j, carry):
            base = (s + NS * j) * CHUNK
            pltpu.sync_copy(idx_hbm.at[pl.ds(base, CHUNK)], idx_v)
            pltpu.async_copy(nodes_hbm.at[idx_v], rows_v, sem).wait()
            pltpu.sync_copy(rows_v, out_hbm.at[pl.ds(base, CHUNK)])
            return carry
        lax.fori_loop(0, nj, body, 0)

    @pl.when(c == 0)
    def _():
        run(send_hbm, sent_out)

    @pl.when(c == 1)
    def _():
        run(recv_hbm, recv_out)


@functools.cache
def _sc_gather_kernel():
    mesh = plsc.VectorSubcoreMesh(core_axis_name="c", subcore_axis_name="s",
                                  num_cores=NC, num_subcores=NS)
    return pl.kernel(
        _gather_body,
        out_type=(
            jax.ShapeDtypeStruct((E, D), jnp.float32),
            jax.ShapeDtypeStruct((E, D), jnp.float32),
        ),
        mesh=mesh,
        scratch_types=(
            pltpu.VMEM((CHUNK,), jnp.int32),
            pltpu.VMEM((CHUNK, D), jnp.float32),
            pltpu.SemaphoreType.DMA,
        ),
    )


def _sc_gather(nodes, senders, receivers):
    return _sc_gather_kernel()(nodes, senders, receivers)


def _scatter_body(wsend_hbm, wrecv_hbm, send_hbm, recv_hbm, zeros_hbm,
                  sent_out, recv_out, idx_v, rows_v, acc, sem):
    del sem
    c = lax.axis_index("c")
    s = lax.axis_index("s")
    nj = jnp.where(s < EXTRA, NJ + 1, NJ)
    rbase = s * ROWS_PER_TILE

    pltpu.sync_copy(zeros_hbm.at[pl.ds(rbase, ROWS_PER_TILE)],
                    acc.at[pl.ds(rbase, ROWS_PER_TILE)])

    @pl.when(s == NS - 1)
    def _():
        pltpu.sync_copy(zeros_hbm.at[pl.ds(NS * ROWS_PER_TILE, ROWS_REM)],
                        acc.at[pl.ds(NS * ROWS_PER_TILE, ROWS_REM)])

    plsc.subcore_barrier()

    def run(idx_hbm, upd_hbm):
        def body(j, carry):
            base = (s + NS * j) * CHUNK
            pltpu.sync_copy(idx_hbm.at[pl.ds(base, CHUNK)], idx_v)
            pltpu.sync_copy(upd_hbm.at[pl.ds(base, CHUNK)], rows_v)
            pltpu.sync_copy(rows_v, acc.at[idx_v], add=True)
            return carry
        lax.fori_loop(0, nj, body, 0)

    @pl.when(c == 0)
    def _():
        run(send_hbm, wsend_hbm)

    @pl.when(c == 1)
    def _():
        run(recv_hbm, wrecv_hbm)

    plsc.subcore_barrier()

    def writeback(out_hbm):
        pltpu.sync_copy(acc.at[pl.ds(rbase, ROWS_PER_TILE)],
                        out_hbm.at[pl.ds(rbase, ROWS_PER_TILE)])

        @pl.when(s == NS - 1)
        def _():
            pltpu.sync_copy(acc.at[pl.ds(NS * ROWS_PER_TILE, ROWS_REM)],
                            out_hbm.at[pl.ds(NS * ROWS_PER_TILE, ROWS_REM)])

    @pl.when(c == 0)
    def _():
        writeback(sent_out)

    @pl.when(c == 1)
    def _():
        writeback(recv_out)


@functools.cache
def _sc_scatter_kernel():
    mesh = plsc.VectorSubcoreMesh(core_axis_name="c", subcore_axis_name="s",
                                  num_cores=NC, num_subcores=NS)
    return pl.kernel(
        _scatter_body,
        out_type=(
            jax.ShapeDtypeStruct((N, D), jnp.float32),
            jax.ShapeDtypeStruct((N, D), jnp.float32),
        ),
        mesh=mesh,
        scratch_types=(
            pltpu.VMEM((CHUNK,), jnp.int32),
            pltpu.VMEM((CHUNK, D), jnp.float32),
            pltpu.VMEM_SHARED((N, D), jnp.float32),
            pltpu.SemaphoreType.DMA,
        ),
    )


def _sc_scatter(wsend, wrecv, senders, receivers, zeros):
    return _sc_scatter_kernel()(wsend, wrecv, senders, receivers, zeros)


def _ssp(x):
    return jnp.maximum(x, 0.0) + jnp.log1p(jnp.exp(-jnp.abs(x))) - _LOG2


def _dot(a, b):
    return jax.lax.dot_general(a, b, (((1,), (0,)), ((), ())),
                               precision=jax.lax.Precision.HIGHEST,
                               preferred_element_type=jnp.float32)


def _mlp_ln(h, W2, b2, W3, b3, gamma, beta):
    h = _ssp(h)
    h = _ssp(_dot(h, W2) + b2)
    h = _dot(h, W3) + b3
    mu = jnp.mean(h, axis=1, keepdims=True)
    var = jnp.mean((h - mu) ** 2, axis=1, keepdims=True)
    h = (h - mu) * jax.lax.rsqrt(var + 1e-5)
    return h * gamma + beta


def _edge_block(e_ref, sa_ref, ra_ref,
                W1e_ref, W1s_ref, W1r_ref, b1_ref, W2_ref, b2_ref,
                W3_ref, b3_ref, g_ref, bt_ref,
                wr_ref, br_ref, ws_ref, bs_ref,
                new_e_ref, wsend_ref, wrecv_ref):
    x = e_ref[...]
    h = (_dot(x, W1e_ref[...]) + _dot(sa_ref[...], W1s_ref[...])
         + _dot(ra_ref[...], W1r_ref[...]) + b1_ref[...])
    u = _mlp_ln(h, W2_ref[...], b2_ref[...], W3_ref[...], b3_ref[...],
                g_ref[...], bt_ref[...])
    attn_r = jax.nn.sigmoid(
        jnp.sum(x * wr_ref[...], axis=1, keepdims=True) + br_ref[...])
    attn_s = jax.nn.sigmoid(
        jnp.sum(x * ws_ref[...], axis=1, keepdims=True) + bs_ref[...])
    new_e_ref[...] = x + u
    wsend_ref[...] = u * attn_s
    wrecv_ref[...] = u * attn_r


def _node_block(n_ref, ragg_ref, sagg_ref,
                W1n_ref, W1r_ref, W1s_ref, b1_ref, W2_ref, b2_ref,
                W3_ref, b3_ref, g_ref, bt_ref,
                out_ref):
    x = n_ref[...]
    h = (_dot(x, W1n_ref[...]) + _dot(ragg_ref[...], W1r_ref[...])
         + _dot(sagg_ref[...], W1s_ref[...]) + b1_ref[...])
    u = _mlp_ln(h, W2_ref[...], b2_ref[...], W3_ref[...], b3_ref[...],
                g_ref[...], bt_ref[...])
    out_ref[...] = x + u


def _row_spec(rows, i_dim=0):
    return pl.BlockSpec((rows, D), lambda i: (i, 0))


def _full_spec(shape):
    return pl.BlockSpec(shape, lambda i: tuple(0 for _ in shape))


BE = 2000
BN = 1000


def _edge_call(edges, sent_attr, recv_attr, p, wr, br, ws, bs):
    W1, W2, W3 = p["Ws"]
    b1, b2, b3 = p["bs"]
    W1e, W1s, W1r = W1[:D], W1[D:2 * D], W1[2 * D:]
    row = lambda v: v.reshape(1, -1)
    weights = (W1e, W1s, W1r, row(b1), W2, row(b2), W3, row(b3),
               row(p["gamma"]), row(p["beta"]),
               wr.reshape(1, D), br.reshape(1, 1),
               ws.reshape(1, D), bs.reshape(1, 1))
    w_specs = [_full_spec(w.shape) for w in weights]
    out_shape = (
        jax.ShapeDtypeStruct((E, D), jnp.float32),
        jax.ShapeDtypeStruct((E, D), jnp.float32),
        jax.ShapeDtypeStruct((E, D), jnp.float32),
    )
    return pl.pallas_call(
        _edge_block,
        grid=(E // BE,),
        in_specs=[_row_spec(BE)] * 3 + w_specs,
        out_specs=(_row_spec(BE),) * 3,
        out_shape=out_shape,
    )(edges, sent_attr, recv_attr, *weights)


def _node_call(nodes, recv_agg, sent_agg, p):
    W1, W2, W3 = p["Ws"]
    b1, b2, b3 = p["bs"]
    W1n, W1r, W1s = W1[:D], W1[D:2 * D], W1[2 * D:]
    row = lambda v: v.reshape(1, -1)
    weights = (W1n, W1r, W1s, row(b1), W2, row(b2), W3, row(b3),
               row(p["gamma"]), row(p["beta"]))
    w_specs = [_full_spec(w.shape) for w in weights]
    return pl.pallas_call(
        _node_block,
        grid=(N // BN,),
        in_specs=[_row_spec(BN)] * 3 + w_specs,
        out_specs=_row_spec(BN),
        out_shape=jax.ShapeDtypeStruct((N, D), jnp.float32),
    )(nodes, recv_agg, sent_agg, *weights)


def kernel(nodes, edges, senders, receivers, cutoff, params):
    del cutoff
    sent_attr, recv_attr = _sc_gather(nodes, senders, receivers)
    new_edges, wsend, wrecv = _edge_call(
        edges, sent_attr, recv_attr, params["edge_mlp"],
        params["w_recv"], params["b_recv"], params["w_send"], params["b_send"])
    zeros = jnp.zeros((N, D), jnp.float32)
    sent_agg, recv_agg = _sc_scatter(wsend, wrecv, senders, receivers, zeros)
    new_nodes = _node_call(nodes, recv_agg, sent_agg, params["node_mlp"])
    return (new_nodes, new_edges)


# speedup vs baseline: 3.1477x; 1.7397x over previous
import functools

import jax
import jax.numpy as jnp
from jax import lax
from jax.experimental import pallas as pl
from jax.experimental.pallas import tpu as pltpu
from jax.experimental.pallas import tpu_sc as plsc

N = 10000
E = 320000
D = 128

NC = 2
NS = 16
CHUNK = 128
NCHUNKS = E // CHUNK
NJ = NCHUNKS // NS
EXTRA = NCHUNKS % NS
ROWS_PER_TILE = 624
ROWS_REM = N - NS * ROWS_PER_TILE

_LOG2 = 0.6931471805599453


def _gather_body(nodes_hbm, send_hbm, recv_hbm, sent_out, recv_out,
                 idx_v, rows_v, sem):
    c = lax.axis_index("c")
    s = lax.axis_index("s")
    nj = jnp.where(s < EXTRA, NJ + 1, NJ)

    def run(idx_hbm, out_hbm):
        def body(j, carry):
            base = (s + NS * j) * CHUNK
            pltpu.sync_copy(idx_hbm.at[pl.ds(base, CHUNK)], idx_v)
            pltpu.async_copy(nodes_hbm.at[idx_v], rows_v, sem).wait()
            pltpu.sync_copy(rows_v, out_hbm.at[pl.ds(base, CHUNK)])
            return carry
        lax.fori_loop(0, nj, body, 0)

    @pl.when(c == 0)
    def _():
        run(send_hbm, sent_out)

    @pl.when(c == 1)
    def _():
        run(recv_hbm, recv_out)


@functools.cache
def _sc_gather_kernel():
    mesh = plsc.VectorSubcoreMesh(core_axis_name="c", subcore_axis_name="s",
                                  num_cores=NC, num_subcores=NS)
    return pl.kernel(
        _gather_body,
        out_type=(
            jax.ShapeDtypeStruct((E, D), jnp.float32),
            jax.ShapeDtypeStruct((E, D), jnp.float32),
        ),
        mesh=mesh,
        scratch_types=(
            pltpu.VMEM((CHUNK,), jnp.int32),
            pltpu.VMEM((CHUNK, D), jnp.float32),
            pltpu.SemaphoreType.DMA,
        ),
    )


def _sc_gather(nodes, senders, receivers):
    return _sc_gather_kernel()(nodes, senders, receivers)


def _scatter_body(wsend_hbm, wrecv_hbm, send_hbm, recv_hbm, zeros_hbm,
                  sent_out, recv_out, idx_v, rows_v, acc, sem):
    del sem
    c = lax.axis_index("c")
    s = lax.axis_index("s")
    nj = jnp.where(s < EXTRA, NJ + 1, NJ)
    rbase = s * ROWS_PER_TILE

    pltpu.sync_copy(zeros_hbm.at[pl.ds(rbase, ROWS_PER_TILE)],
                    acc.at[pl.ds(rbase, ROWS_PER_TILE)])

    @pl.when(s == NS - 1)
    def _():
        pltpu.sync_copy(zeros_hbm.at[pl.ds(NS * ROWS_PER_TILE, ROWS_REM)],
                        acc.at[pl.ds(NS * ROWS_PER_TILE, ROWS_REM)])

    plsc.subcore_barrier()

    def run(idx_hbm, upd_hbm):
        def body(j, carry):
            base = (s + NS * j) * CHUNK
            pltpu.sync_copy(idx_hbm.at[pl.ds(base, CHUNK)], idx_v)
            pltpu.sync_copy(upd_hbm.at[pl.ds(base, CHUNK)], rows_v)
            pltpu.sync_copy(rows_v, acc.at[idx_v], add=True)
            return carry
        lax.fori_loop(0, nj, body, 0)

    @pl.when(c == 0)
    def _():
        run(send_hbm, wsend_hbm)

    @pl.when(c == 1)
    def _():
        run(recv_hbm, wrecv_hbm)

    plsc.subcore_barrier()

    def writeback(out_hbm):
        pltpu.sync_copy(acc.at[pl.ds(rbase, ROWS_PER_TILE)],
                        out_hbm.at[pl.ds(rbase, ROWS_PER_TILE)])

        @pl.when(s == NS - 1)
        def _():
            pltpu.sync_copy(acc.at[pl.ds(NS * ROWS_PER_TILE, ROWS_REM)],
                            out_hbm.at[pl.ds(NS * ROWS_PER_TILE, ROWS_REM)])

    @pl.when(c == 0)
    def _():
        writeback(sent_out)

    @pl.when(c == 1)
    def _():
        writeback(recv_out)


@functools.cache
def _sc_scatter_kernel():
    mesh = plsc.VectorSubcoreMesh(core_axis_name="c", subcore_axis_name="s",
                                  num_cores=NC, num_subcores=NS)
    return pl.kernel(
        _scatter_body,
        out_type=(
            jax.ShapeDtypeStruct((N, D), jnp.float32),
            jax.ShapeDtypeStruct((N, D), jnp.float32),
        ),
        mesh=mesh,
        scratch_types=(
            pltpu.VMEM((CHUNK,), jnp.int32),
            pltpu.VMEM((CHUNK, D), jnp.float32),
            pltpu.VMEM_SHARED((N, D), jnp.float32),
            pltpu.SemaphoreType.DMA,
        ),
    )


def _sc_scatter(wsend, wrecv, senders, receivers, zeros):
    return _sc_scatter_kernel()(wsend, wrecv, senders, receivers, zeros)


def _ssp(x):
    return jnp.maximum(x, 0.0) + jnp.log1p(jnp.exp(-jnp.abs(x))) - _LOG2


def _dot(a, b):
    return jax.lax.dot_general(a.astype(jnp.bfloat16), b.astype(jnp.bfloat16),
                               (((1,), (0,)), ((), ())),
                               preferred_element_type=jnp.float32)


def _mlp_ln(h, W2, b2, W3, b3, gamma, beta):
    h = _ssp(h)
    h = _ssp(_dot(h, W2) + b2)
    h = _dot(h, W3) + b3
    mu = jnp.mean(h, axis=1, keepdims=True)
    var = jnp.mean((h - mu) ** 2, axis=1, keepdims=True)
    h = (h - mu) * jax.lax.rsqrt(var + 1e-5)
    return h * gamma + beta


def _edge_block(e_ref, sa_ref, ra_ref,
                W1e_ref, W1s_ref, W1r_ref, b1_ref, W2_ref, b2_ref,
                W3_ref, b3_ref, g_ref, bt_ref,
                wr_ref, br_ref, ws_ref, bs_ref,
                new_e_ref, wsend_ref, wrecv_ref):
    x = e_ref[...]
    h = (_dot(x, W1e_ref[...]) + _dot(sa_ref[...], W1s_ref[...])
         + _dot(ra_ref[...], W1r_ref[...]) + b1_ref[...])
    u = _mlp_ln(h, W2_ref[...], b2_ref[...], W3_ref[...], b3_ref[...],
                g_ref[...], bt_ref[...])
    attn_r = jax.nn.sigmoid(
        jnp.sum(x * wr_ref[...], axis=1, keepdims=True) + br_ref[...])
    attn_s = jax.nn.sigmoid(
        jnp.sum(x * ws_ref[...], axis=1, keepdims=True) + bs_ref[...])
    new_e_ref[...] = x + u
    wsend_ref[...] = u * attn_s
    wrecv_ref[...] = u * attn_r


def _node_block(n_ref, ragg_ref, sagg_ref,
                W1n_ref, W1r_ref, W1s_ref, b1_ref, W2_ref, b2_ref,
                W3_ref, b3_ref, g_ref, bt_ref,
                out_ref):
    x = n_ref[...]
    h = (_dot(x, W1n_ref[...]) + _dot(ragg_ref[...], W1r_ref[...])
         + _dot(sagg_ref[...], W1s_ref[...]) + b1_ref[...])
    u = _mlp_ln(h, W2_ref[...], b2_ref[...], W3_ref[...], b3_ref[...],
                g_ref[...], bt_ref[...])
    out_ref[...] = x + u


def _row_spec(rows, i_dim=0):
    return pl.BlockSpec((rows, D), lambda i: (i, 0))


def _full_spec(shape):
    return pl.BlockSpec(shape, lambda i: tuple(0 for _ in shape))


BE = 2000
BN = 1000


def _edge_call(edges, sent_attr, recv_attr, p, wr, br, ws, bs):
    W1, W2, W3 = p["Ws"]
    b1, b2, b3 = p["bs"]
    W1e, W1s, W1r = W1[:D], W1[D:2 * D], W1[2 * D:]
    row = lambda v: v.reshape(1, -1)
    weights = (W1e, W1s, W1r, row(b1), W2, row(b2), W3, row(b3),
               row(p["gamma"]), row(p["beta"]),
               wr.reshape(1, D), br.reshape(1, 1),
               ws.reshape(1, D), bs.reshape(1, 1))
    w_specs = [_full_spec(w.shape) for w in weights]
    out_shape = (
        jax.ShapeDtypeStruct((E, D), jnp.float32),
        jax.ShapeDtypeStruct((E, D), jnp.float32),
        jax.ShapeDtypeStruct((E, D), jnp.float32),
    )
    return pl.pallas_call(
        _edge_block,
        grid=(E // BE,),
        in_specs=[_row_spec(BE)] * 3 + w_specs,
        out_specs=(_row_spec(BE),) * 3,
        out_shape=out_shape,
    )(edges, sent_attr, recv_attr, *weights)


def _node_call(nodes, recv_agg, sent_agg, p):
    W1, W2, W3 = p["Ws"]
    b1, b2, b3 = p["bs"]
    W1n, W1r, W1s = W1[:D], W1[D:2 * D], W1[2 * D:]
    row = lambda v: v.reshape(1, -1)
    weights = (W1n, W1r, W1s, row(b1), W2, row(b2), W3, row(b3),
               row(p["gamma"]), row(p["beta"]))
    w_specs = [_full_spec(w.shape) for w in weights]
    return pl.pallas_call(
        _node_block,
        grid=(N // BN,),
        in_specs=[_row_spec(BN)] * 3 + w_specs,
        out_specs=_row_spec(BN),
        out_shape=jax.ShapeDtypeStruct((N, D), jnp.float32),
    )(nodes, recv_agg, sent_agg, *weights)


def kernel(nodes, edges, senders, receivers, cutoff, params):
    del cutoff
    sent_attr, recv_attr = _sc_gather(nodes, senders, receivers)
    new_edges, wsend, wrecv = _edge_call(
        edges, sent_attr, recv_attr, params["edge_mlp"],
        params["w_recv"], params["b_recv"], params["w_send"], params["b_send"])
    zeros = jnp.zeros((N, D), jnp.float32)
    sent_agg, recv_agg = _sc_scatter(wsend, wrecv, senders, receivers, zeros)
    new_nodes = _node_call(nodes, recv_agg, sent_agg, params["node_mlp"])
    return (new_nodes, new_edges)


# speedup vs baseline: 4.0409x; 1.2838x over previous
import functools

import jax
import jax.numpy as jnp
from jax import lax
from jax.experimental import pallas as pl
from jax.experimental.pallas import tpu as pltpu
from jax.experimental.pallas import tpu_sc as plsc

N = 10000
E = 320000
D = 128

NC = 2
NS = 16
CHUNK = 128
NCHUNKS = E // CHUNK
CPT = 160
CPT_LAST = NCHUNKS - (NS - 1) * CPT
CPT_LAST_LOAD = 104
NCHUNKS_PAD = (NS - 1) * CPT + CPT_LAST_LOAD
ROWS_PER_TILE = 624
ROWS_REM = N - NS * ROWS_PER_TILE

_LOG2 = 0.6931471805599453


def _load_tile_indices(idx2d_hbm, idx_all, s, cb):
    @pl.when(s < NS - 1)
    def _():
        pltpu.sync_copy(idx2d_hbm.at[pl.ds(cb, CPT)], idx_all)

    @pl.when(s == NS - 1)
    def _():
        pltpu.sync_copy(idx2d_hbm.at[pl.ds(cb, CPT_LAST_LOAD)],
                        idx_all.at[pl.ds(0, CPT_LAST_LOAD)])


def _gather_body(nodes_hbm, send2d, recv2d, sent_out, recv_out,
                 idx_all, rows0, rows1, gsem0, gsem1, wsem0, wsem1):
    c = lax.axis_index("c")
    s = lax.axis_index("s")
    cb = s * CPT
    npairs = jnp.where(s == NS - 1, CPT_LAST // 2, CPT // 2)

    def run(idx2d_hbm, out_hbm):
        _load_tile_indices(idx2d_hbm, idx_all, s, cb)

        def gather_start(j, buf, sem):
            pltpu.async_copy(nodes_hbm.at[idx_all.at[j]], buf, sem)

        def gather_wait(j, buf, sem):
            pltpu.make_async_copy(nodes_hbm.at[idx_all.at[j]], buf, sem).wait()

        def write_start(j, buf, sem):
            pltpu.async_copy(buf, out_hbm.at[pl.ds((cb + j) * CHUNK, CHUNK)],
                             sem)

        def write_wait(sem):
            pltpu.make_async_copy(rows0, out_hbm.at[pl.ds(0, CHUNK)],
                                  sem).wait()

        def body(t, carry):
            j0 = 2 * t
            j1 = j0 + 1

            @pl.when(t > 0)
            def _():
                write_wait(wsem0)

            gather_start(j0, rows0, gsem0)

            @pl.when(t > 0)
            def _():
                write_wait(wsem1)

            gather_start(j1, rows1, gsem1)
            gather_wait(j0, rows0, gsem0)
            write_start(j0, rows0, wsem0)
            gather_wait(j1, rows1, gsem1)
            write_start(j1, rows1, wsem1)
            return carry

        lax.fori_loop(0, npairs, body, 0)
        write_wait(wsem0)
        write_wait(wsem1)

    @pl.when(c == 0)
    def _():
        run(send2d, sent_out)

    @pl.when(c == 1)
    def _():
        run(recv2d, recv_out)


@functools.cache
def _sc_gather_kernel():
    mesh = plsc.VectorSubcoreMesh(core_axis_name="c", subcore_axis_name="s",
                                  num_cores=NC, num_subcores=NS)
    return pl.kernel(
        _gather_body,
        out_type=(
            jax.ShapeDtypeStruct((E, D), jnp.float32),
            jax.ShapeDtypeStruct((E, D), jnp.float32),
        ),
        mesh=mesh,
        scratch_types=(
            pltpu.VMEM((CPT, CHUNK), jnp.int32),
            pltpu.VMEM((CHUNK, D), jnp.float32),
            pltpu.VMEM((CHUNK, D), jnp.float32),
            pltpu.SemaphoreType.DMA,
            pltpu.SemaphoreType.DMA,
            pltpu.SemaphoreType.DMA,
            pltpu.SemaphoreType.DMA,
        ),
    )


def _sc_gather(nodes, senders2d, receivers2d):
    return _sc_gather_kernel()(nodes, senders2d, receivers2d)


def _scatter_body(wsend_hbm, wrecv_hbm, send_hbm, recv_hbm, zeros_hbm,
                  sent_out, recv_out, idx0, idx1, rows0, rows1, acc,
                  lsem0, lsem1, ssem0, ssem1):
    c = lax.axis_index("c")
    s = lax.axis_index("s")
    cb = s * CPT
    npairs = jnp.where(s == NS - 1, CPT_LAST // 2, CPT // 2)
    rbase = s * ROWS_PER_TILE

    pltpu.sync_copy(zeros_hbm.at[pl.ds(rbase, ROWS_PER_TILE)],
                    acc.at[pl.ds(rbase, ROWS_PER_TILE)])

    @pl.when(s == NS - 1)
    def _():
        pltpu.sync_copy(zeros_hbm.at[pl.ds(NS * ROWS_PER_TILE, ROWS_REM)],
                        acc.at[pl.ds(NS * ROWS_PER_TILE, ROWS_REM)])

    plsc.subcore_barrier()

    def run(idx_hbm, upd_hbm):
        def load_start(j, ibuf, rbuf, sem):
            base = (cb + j) * CHUNK
            pltpu.async_copy(idx_hbm.at[pl.ds(base, CHUNK)], ibuf, sem)
            pltpu.async_copy(upd_hbm.at[pl.ds(base, CHUNK)], rbuf, sem)

        def load_wait(j, ibuf, rbuf, sem):
            base = (cb + j) * CHUNK
            pltpu.make_async_copy(idx_hbm.at[pl.ds(base, CHUNK)], ibuf,
                                  sem).wait()
            pltpu.make_async_copy(upd_hbm.at[pl.ds(base, CHUNK)], rbuf,
                                  sem).wait()

        def scat_start(ibuf, rbuf, sem):
            pltpu.async_copy(rbuf, acc.at[ibuf], sem, add=True)

        def scat_wait(sem):
            pltpu.make_async_copy(rows0, acc.at[idx0], sem).wait()

        def body(t, carry):
            j0 = 2 * t
            j1 = j0 + 1

            @pl.when(t > 0)
            def _():
                scat_wait(ssem0)

            load_start(j0, idx0, rows0, lsem0)

            @pl.when(t > 0)
            def _():
                scat_wait(ssem1)

            load_start(j1, idx1, rows1, lsem1)
            load_wait(j0, idx0, rows0, lsem0)
            scat_start(idx0, rows0, ssem0)
            load_wait(j1, idx1, rows1, lsem1)
            scat_start(idx1, rows1, ssem1)
            return carry

        lax.fori_loop(0, npairs, body, 0)
        scat_wait(ssem0)
        scat_wait(ssem1)

    @pl.when(c == 0)
    def _():
        run(send_hbm, wsend_hbm)

    @pl.when(c == 1)
    def _():
        run(recv_hbm, wrecv_hbm)

    plsc.subcore_barrier()

    def writeback(out_hbm):
        pltpu.sync_copy(acc.at[pl.ds(rbase, ROWS_PER_TILE)],
                        out_hbm.at[pl.ds(rbase, ROWS_PER_TILE)])

        @pl.when(s == NS - 1)
        def _():
            pltpu.sync_copy(acc.at[pl.ds(NS * ROWS_PER_TILE, ROWS_REM)],
                            out_hbm.at[pl.ds(NS * ROWS_PER_TILE, ROWS_REM)])

    @pl.when(c == 0)
    def _():
        writeback(sent_out)

    @pl.when(c == 1)
    def _():
        writeback(recv_out)


@functools.cache
def _sc_scatter_kernel():
    mesh = plsc.VectorSubcoreMesh(core_axis_name="c", subcore_axis_name="s",
                                  num_cores=NC, num_subcores=NS)
    return pl.kernel(
        _scatter_body,
        out_type=(
            jax.ShapeDtypeStruct((N, D), jnp.float32),
            jax.ShapeDtypeStruct((N, D), jnp.float32),
        ),
        mesh=mesh,
        scratch_types=(
            pltpu.VMEM((CHUNK,), jnp.int32),
            pltpu.VMEM((CHUNK,), jnp.int32),
            pltpu.VMEM((CHUNK, D), jnp.float32),
            pltpu.VMEM((CHUNK, D), jnp.float32),
            pltpu.VMEM_SHARED((N, D), jnp.float32),
            pltpu.SemaphoreType.DMA,
            pltpu.SemaphoreType.DMA,
            pltpu.SemaphoreType.DMA,
            pltpu.SemaphoreType.DMA,
        ),
    )


def _sc_scatter(wsend, wrecv, senders, receivers, zeros):
    return _sc_scatter_kernel()(wsend, wrecv, senders, receivers, zeros)


def _ssp(x):
    return jnp.maximum(x, 0.0) + jnp.log1p(jnp.exp(-jnp.abs(x))) - _LOG2


def _dot(a, b):
    return jax.lax.dot_general(a.astype(jnp.bfloat16), b.astype(jnp.bfloat16),
                               (((1,), (0,)), ((), ())),
                               preferred_element_type=jnp.float32)


def _mlp_ln(h, W2, b2, W3, b3, gamma, beta):
    h = _ssp(h)
    h = _ssp(_dot(h, W2) + b2)
    h = _dot(h, W3) + b3
    mu = jnp.mean(h, axis=1, keepdims=True)
    var = jnp.mean((h - mu) ** 2, axis=1, keepdims=True)
    h = (h - mu) * jax.lax.rsqrt(var + 1e-5)
    return h * gamma + beta


def _edge_block(e_ref, sa_ref, ra_ref,
                W1e_ref, W1s_ref, W1r_ref, b1_ref, W2_ref, b2_ref,
                W3_ref, b3_ref, g_ref, bt_ref,
                wr_ref, br_ref, ws_ref, bs_ref,
                new_e_ref, wsend_ref, wrecv_ref):
    x = e_ref[...]
    h = (_dot(x, W1e_ref[...]) + _dot(sa_ref[...], W1s_ref[...])
         + _dot(ra_ref[...], W1r_ref[...]) + b1_ref[...])
    u = _mlp_ln(h, W2_ref[...], b2_ref[...], W3_ref[...], b3_ref[...],
                g_ref[...], bt_ref[...])
    attn_r = jax.nn.sigmoid(
        jnp.sum(x * wr_ref[...], axis=1, keepdims=True) + br_ref[...])
    attn_s = jax.nn.sigmoid(
        jnp.sum(x * ws_ref[...], axis=1, keepdims=True) + bs_ref[...])
    new_e_ref[...] = x + u
    wsend_ref[...] = u * attn_s
    wrecv_ref[...] = u * attn_r


def _node_block(n_ref, ragg_ref, sagg_ref,
                W1n_ref, W1r_ref, W1s_ref, b1_ref, W2_ref, b2_ref,
                W3_ref, b3_ref, g_ref, bt_ref,
                out_ref):
    x = n_ref[...]
    h = (_dot(x, W1n_ref[...]) + _dot(ragg_ref[...], W1r_ref[...])
         + _dot(sagg_ref[...], W1s_ref[...]) + b1_ref[...])
    u = _mlp_ln(h, W2_ref[...], b2_ref[...], W3_ref[...], b3_ref[...],
                g_ref[...], bt_ref[...])
    out_ref[...] = x + u


def _row_spec(rows, i_dim=0):
    return pl.BlockSpec((rows, D), lambda i: (i, 0))


def _full_spec(shape):
    return pl.BlockSpec(shape, lambda i: tuple(0 for _ in shape))


BE = 2000
BN = 1000


def _edge_call(edges, sent_attr, recv_attr, p, wr, br, ws, bs):
    W1, W2, W3 = p["Ws"]
    b1, b2, b3 = p["bs"]
    W1e, W1s, W1r = W1[:D], W1[D:2 * D], W1[2 * D:]
    row = lambda v: v.reshape(1, -1)
    weights = (W1e, W1s, W1r, row(b1), W2, row(b2), W3, row(b3),
               row(p["gamma"]), row(p["beta"]),
               wr.reshape(1, D), br.reshape(1, 1),
               ws.reshape(1, D), bs.reshape(1, 1))
    w_specs = [_full_spec(w.shape) for w in weights]
    out_shape = (
        jax.ShapeDtypeStruct((E, D), jnp.float32),
        jax.ShapeDtypeStruct((E, D), jnp.float32),
        jax.ShapeDtypeStruct((E, D), jnp.float32),
    )
    return pl.pallas_call(
        _edge_block,
        grid=(E // BE,),
        in_specs=[_row_spec(BE)] * 3 + w_specs,
        out_specs=(_row_spec(BE),) * 3,
        out_shape=out_shape,
    )(edges, sent_attr, recv_attr, *weights)


def _node_call(nodes, recv_agg, sent_agg, p):
    W1, W2, W3 = p["Ws"]
    b1, b2, b3 = p["bs"]
    W1n, W1r, W1s = W1[:D], W1[D:2 * D], W1[2 * D:]
    row = lambda v: v.reshape(1, -1)
    weights = (W1n, W1r, W1s, row(b1), W2, row(b2), W3, row(b3),
               row(p["gamma"]), row(p["beta"]))
    w_specs = [_full_spec(w.shape) for w in weights]
    return pl.pallas_call(
        _node_block,
        grid=(N // BN,),
        in_specs=[_row_spec(BN)] * 3 + w_specs,
        out_specs=_row_spec(BN),
        out_shape=jax.ShapeDtypeStruct((N, D), jnp.float32),
    )(nodes, recv_agg, sent_agg, *weights)


def kernel(nodes, edges, senders, receivers, cutoff, params):
    del cutoff
    pad = ((0, NCHUNKS_PAD - NCHUNKS), (0, 0))
    senders2d = jnp.pad(senders.reshape(NCHUNKS, CHUNK), pad)
    receivers2d = jnp.pad(receivers.reshape(NCHUNKS, CHUNK), pad)
    sent_attr, recv_attr = _sc_gather(nodes, senders2d, receivers2d)
    new_edges, wsend, wrecv = _edge_call(
        edges, sent_attr, recv_attr, params["edge_mlp"],
        params["w_recv"], params["b_recv"], params["w_send"], params["b_send"])
    zeros = jnp.zeros((N, D), jnp.float32)
    sent_agg, recv_agg = _sc_scatter(wsend, wrecv, senders, receivers, zeros)
    new_nodes = _node_call(nodes, recv_agg, sent_agg, params["node_mlp"])
    return (new_nodes, new_edges)
